# Initial kernel scaffold; baseline (speedup 1.0000x reference)
import jax
import jax.numpy as jnp
from jax.experimental import pallas as pl


def kernel(x, edge_index, batch, W, b):
    raise NotImplementedError("write your pallas kernel here")


# speedup vs baseline: 19.4188x; 19.4188x over previous
import functools

import jax
import jax.numpy as jnp
from jax import lax
from jax.experimental import pallas as pl
from jax.experimental.pallas import tpu as pltpu
from jax.experimental.pallas import tpu_sc as plsc

NC = 2
NS = 16
NW = NC * NS
L = 128


def _mesh():
    return plsc.VectorSubcoreMesh(
        core_axis_name="c", subcore_axis_name="s", num_cores=NC,
        num_subcores=NS)


def _sc_deg_body(NP, NPT, CH, col_hbm, deg_out, colv, onesv, zbuf, deg_sh):
    c = lax.axis_index("c")
    s = lax.axis_index("s")
    wid = s * NC + c

    def _zloop(i, carry):
        zbuf[pl.ds(i * 16, 16)] = jnp.zeros((16,), jnp.float32)
        return carry
    lax.fori_loop(0, zbuf.shape[0] // 16, _zloop, 0)

    def _oloop(i, carry):
        onesv[pl.ds(i * 16, 16)] = jnp.ones((16,), jnp.float32)
        return carry
    lax.fori_loop(0, L // 16, _oloop, 0)

    pltpu.sync_copy(zbuf.at[pl.ds(0, NPT)], deg_sh.at[pl.ds(s * NPT, NPT)])
    pltpu.sync_copy(col_hbm.at[wid], colv)
    plsc.subcore_barrier()

    def _eloop(j, carry):
        pltpu.sync_copy(onesv, deg_sh.at[colv.at[j]], add=True)
        return carry
    lax.fori_loop(0, CH, _eloop, 0)

    plsc.subcore_barrier()
    pltpu.sync_copy(deg_sh.at[pl.ds(s * NPT, NPT)], zbuf.at[pl.ds(0, NPT)])
    pltpu.sync_copy(zbuf.at[pl.ds(0, NPT)], deg_out.at[c, pl.ds(s * NPT, NPT)])


def _sc_agg_body(NP, NPT, CH, D, row_hbm, col_hbm, g_hbm, zrows_hbm, acc_out,
                 rowv, colv, gbuf, stage, acc_sh, sem):
    c = lax.axis_index("c")
    s = lax.axis_index("s")
    wid = s * NC + c

    pltpu.sync_copy(zrows_hbm, acc_sh.at[pl.ds(s * NPT, NPT)])
    pltpu.sync_copy(row_hbm.at[wid], rowv)
    pltpu.sync_copy(col_hbm.at[wid], colv)
    plsc.subcore_barrier()

    def _eloop(j, carry):
        pltpu.async_copy(g_hbm.at[rowv.at[j]], gbuf, sem).wait()
        pltpu.sync_copy(gbuf, acc_sh.at[colv.at[j]], add=True)
        return carry
    lax.fori_loop(0, CH, _eloop, 0)

    plsc.subcore_barrier()
    nblk = NPT // stage.shape[0]
    def _wloop(q, carry):
        base = s * NPT + q * stage.shape[0]
        pltpu.sync_copy(acc_sh.at[pl.ds(base, stage.shape[0])], stage)
        pltpu.sync_copy(stage, acc_out.at[c, pl.ds(base, stage.shape[0])])
        return carry
    lax.fori_loop(0, nblk, _wloop, 0)


def _tc_g_body(xb, wref, degp, gout):
    h = lax.dot_general(xb[...], wref[...], (((1,), (1,)), ((), ())),
                        preferred_element_type=jnp.float32)
    deg = 1.0 + degp[0] + degp[1]
    dis = lax.rsqrt(deg)
    gout[...] = h * dis[:, None]


def _tc_out_body(accp, gb, degp, bref, outb):
    a = accp[0] + accp[1] + gb[...]
    deg = 1.0 + degp[0] + degp[1]
    dis = lax.rsqrt(deg)
    outb[...] = dis[:, None] * a + bref[...]


def kernel(x, edge_index, batch, W, b):
    N, D_in = x.shape
    D = W.shape[0]
    E = edge_index.shape[1]

    CH = -(-E // (NW * L))
    Epad = NW * CH * L
    NP = -(-(N + 1) // (NS * 80)) * (NS * 80)
    NPT = NP // NS
    BN = 1280
    GRID = NP // BN

    row = edge_index[0]
    col = edge_index[1]
    pad_e = Epad - E
    rowp = jnp.concatenate([row, jnp.zeros((pad_e,), jnp.int32)]
                           ).reshape(NW, CH, L)
    colp = jnp.concatenate([col, jnp.full((pad_e,), N, jnp.int32)]
                           ).reshape(NW, CH, L)
    xp = jnp.concatenate([x, jnp.zeros((NP - N, D_in), x.dtype)])
    zrows = jnp.zeros((NPT, D), jnp.float32)

    deg_call = pl.kernel(
        functools.partial(_sc_deg_body, NP, NPT, CH),
        out_type=jax.ShapeDtypeStruct((NC, NP), jnp.float32),
        mesh=_mesh(),
        scratch_types=[
            pltpu.VMEM((CH, L), jnp.int32),
            pltpu.VMEM((L,), jnp.float32),
            pltpu.VMEM((((NPT + 15) // 16) * 16,), jnp.float32),
            pltpu.VMEM_SHARED((NP,), jnp.float32),
        ],
    )
    degp = deg_call(colp)

    g = pl.pallas_call(
        _tc_g_body,
        grid=(GRID,),
        in_specs=[
            pl.BlockSpec((BN, D_in), lambda i: (i, 0)),
            pl.BlockSpec((D, D_in), lambda i: (0, 0)),
            pl.BlockSpec((NC, BN), lambda i: (0, i)),
        ],
        out_specs=pl.BlockSpec((BN, D), lambda i: (i, 0)),
        out_shape=jax.ShapeDtypeStruct((NP, D), jnp.float32),
    )(xp, W, degp)

    agg_call = pl.kernel(
        functools.partial(_sc_agg_body, NP, NPT, CH, D),
        out_type=jax.ShapeDtypeStruct((NC, NP, D), jnp.float32),
        mesh=_mesh(),
        scratch_types=[
            pltpu.VMEM((CH, L), jnp.int32),
            pltpu.VMEM((CH, L), jnp.int32),
            pltpu.VMEM((L, D), jnp.float32),
            pltpu.VMEM((NPT // 8, D), jnp.float32),
            pltpu.VMEM_SHARED((NP, D), jnp.float32),
            pltpu.SemaphoreType.DMA,
        ],
    )
    accp = agg_call(rowp, colp, g, zrows)

    outp = pl.pallas_call(
        _tc_out_body,
        grid=(GRID,),
        in_specs=[
            pl.BlockSpec((NC, BN, D), lambda i: (0, i, 0)),
            pl.BlockSpec((BN, D), lambda i: (i, 0)),
            pl.BlockSpec((NC, BN), lambda i: (0, i)),
            pl.BlockSpec((1, D), lambda i: (0, 0)),
        ],
        out_specs=pl.BlockSpec((BN, D), lambda i: (i, 0)),
        out_shape=jax.ShapeDtypeStruct((NP, D), jnp.float32),
    )(accp, g, degp, b.reshape(1, D))

    return outp[:N]
